# baseline (speedup 1.0000x reference)
import functools

import jax
import jax.numpy as jnp
from jax import lax
from jax.experimental import pallas as pl
from jax.experimental.pallas import tpu as pltpu
from jax.experimental.pallas import tpu_sc as plsc

_NUM_CORES = 2
_NUM_SUBCORES = 16
_NW = _NUM_CORES * _NUM_SUBCORES

_CHUNK = 64
_D = 300
_B = 4096 * 50
_PER_W = _B // _NW
_NCH = _PER_W // _CHUNK
_TAIL_OFF = 172
_TAIL_IN = 256 - _TAIL_OFF


def _make_gather():
  mesh = plsc.VectorSubcoreMesh(core_axis_name="c", subcore_axis_name="s")

  @functools.partial(
      pl.kernel,
      mesh=mesh,
      compiler_params=pltpu.CompilerParams(needs_layout_passes=False),
      out_type=jax.ShapeDtypeStruct((_B, _D), jnp.float32),
      scratch_types=[
          pltpu.VMEM((_NCH, _CHUNK), jnp.int32),
          pltpu.VMEM((_CHUNK, _D), jnp.float32),
          pltpu.VMEM((_CHUNK, _D), jnp.float32),
          pltpu.VMEM((_CHUNK, 128), jnp.float32),
          pltpu.VMEM((_CHUNK, 128), jnp.float32),
          pltpu.SemaphoreType.DMA,
          pltpu.SemaphoreType.DMA,
      ],
  )
  def gather_kernel(idx_hbm, table_hbm, tail_hbm, out_hbm,
                    idx_v, buf0, buf1, tl0, tl1, sem0, sem1):
    wid = lax.axis_index("s") * _NUM_CORES + lax.axis_index("c")
    base = wid * _PER_W

    pltpu.sync_copy(idx_hbm.at[wid], idx_v)

    def start(j, buf, tl, sem):
      idx = idx_v.at[j]
      pltpu.make_async_copy(
          table_hbm.at[idx, pl.ds(0, 128)],
          buf.at[:, pl.ds(0, 128)], sem).start()
      pltpu.make_async_copy(
          table_hbm.at[idx, pl.ds(128, 128)],
          buf.at[:, pl.ds(128, 128)], sem).start()
      pltpu.make_async_copy(tail_hbm.at[idx], tl, sem).start()

    def wait(buf, tl, sem):
      pltpu.make_async_copy(
          table_hbm.at[idx_v.at[0], pl.ds(0, 128)],
          buf.at[:, pl.ds(0, 128)], sem).wait()
      pltpu.make_async_copy(
          table_hbm.at[idx_v.at[0], pl.ds(128, 128)],
          buf.at[:, pl.ds(128, 128)], sem).wait()
      pltpu.make_async_copy(tail_hbm.at[idx_v.at[0]], tl, sem).wait()

    def compact_and_store(j, buf, tl):
      cols = 284 + lax.iota(jnp.int32, 16)
      def row(i, _):
        buf[i, pl.ds(256, 16)] = tl[i, pl.ds(_TAIL_IN, 16)]
        buf[i, pl.ds(272, 16)] = tl[i, pl.ds(_TAIL_IN + 16, 16)]
        rows = jnp.full((16,), i, jnp.int32)
        plsc.store_scatter(buf, [rows, cols], tl[i, pl.ds(_TAIL_IN + 28, 16)])
        return ()
      lax.fori_loop(0, _CHUNK, row, ())
      pltpu.sync_copy(buf, out_hbm.at[pl.ds(base + j * _CHUNK, _CHUNK)])

    start(0, buf0, tl0, sem0)
    start(1, buf1, tl1, sem1)

    def body(i, _):
      j0 = 2 * i
      j1 = 2 * i + 1

      wait(buf0, tl0, sem0)
      compact_and_store(j0, buf0, tl0)

      @pl.when(j0 + 2 < _NCH)
      def _():
        start(j0 + 2, buf0, tl0, sem0)

      wait(buf1, tl1, sem1)
      compact_and_store(j1, buf1, tl1)

      @pl.when(j1 + 2 < _NCH)
      def _():
        start(j1 + 2, buf1, tl1, sem1)

      return ()

    lax.fori_loop(0, _NCH // 2, body, ())

  return gather_kernel


_gather = _make_gather()

_TAIL_ROWS = 1024


def _tail_body(t_ref, o_ref):
  o_ref[...] = t_ref[:, _TAIL_OFF:_D]


def _make_tail(n_rows):
  grid = (n_rows + _TAIL_ROWS - 1) // _TAIL_ROWS
  return pl.pallas_call(
      _tail_body,
      grid=(grid,),
      in_specs=[pl.BlockSpec((_TAIL_ROWS, _D), lambda i: (i, 0))],
      out_specs=pl.BlockSpec((_TAIL_ROWS, _D - _TAIL_OFF), lambda i: (i, 0)),
      out_shape=jax.ShapeDtypeStruct((n_rows, _D - _TAIL_OFF), jnp.float32),
  )


def kernel(x, table):
  idx = x.astype(jnp.int32).reshape(_NW, _NCH, _CHUNK)
  tail = _make_tail(table.shape[0])(table)
  out = _gather(idx, table, tail)
  return out.reshape(x.shape[0], x.shape[1], _D)


# speedup vs baseline: 1.3343x; 1.3343x over previous
import functools

import jax
import jax.numpy as jnp
from jax import lax
from jax.experimental import pallas as pl
from jax.experimental.pallas import tpu as pltpu
from jax.experimental.pallas import tpu_sc as plsc

_NUM_CORES = 2
_NUM_SUBCORES = 16
_NW = _NUM_CORES * _NUM_SUBCORES

_A = 4096
_S = 50
_D = 300
_ROWS_W = _A // _NW
_TAIL_OFF = 172
_TAIL_IN = 256 - _TAIL_OFF


def _make_gather():
  mesh = plsc.VectorSubcoreMesh(core_axis_name="c", subcore_axis_name="s")

  @functools.partial(
      pl.kernel,
      mesh=mesh,
      compiler_params=pltpu.CompilerParams(needs_layout_passes=False),
      out_type=jax.ShapeDtypeStruct((_A, _S, _D), jnp.float32),
      scratch_types=[
          pltpu.VMEM((_ROWS_W, _S), jnp.int32),
          pltpu.VMEM((_S, _D), jnp.float32),
          pltpu.VMEM((_S, _D), jnp.float32),
          pltpu.VMEM((_S, 128), jnp.float32),
          pltpu.VMEM((_S, 128), jnp.float32),
          pltpu.SemaphoreType.DMA,
          pltpu.SemaphoreType.DMA,
      ],
  )
  def gather_kernel(idx_hbm, table_hbm, tail_hbm, out_hbm,
                    idx_v, buf0, buf1, tl0, tl1, sem0, sem1):
    wid = lax.axis_index("s") * _NUM_CORES + lax.axis_index("c")
    base = wid * _ROWS_W

    pltpu.sync_copy(idx_hbm.at[pl.ds(base, _ROWS_W)], idx_v)

    def start(j, buf, tl, sem):
      idx = idx_v.at[j]
      pltpu.make_async_copy(
          table_hbm.at[idx, pl.ds(0, 128)],
          buf.at[:, pl.ds(0, 128)], sem).start()
      pltpu.make_async_copy(
          table_hbm.at[idx, pl.ds(128, 128)],
          buf.at[:, pl.ds(128, 128)], sem).start()
      pltpu.make_async_copy(tail_hbm.at[idx], tl, sem).start()

    def wait(buf, tl, sem):
      pltpu.make_async_copy(
          table_hbm.at[idx_v.at[0], pl.ds(0, 128)],
          buf.at[:, pl.ds(0, 128)], sem).wait()
      pltpu.make_async_copy(
          table_hbm.at[idx_v.at[0], pl.ds(128, 128)],
          buf.at[:, pl.ds(128, 128)], sem).wait()
      pltpu.make_async_copy(tail_hbm.at[idx_v.at[0]], tl, sem).wait()

    cols = 284 + lax.iota(jnp.int32, 16)

    def compact_and_store(j, buf, tl):
      def row(i, _):
        buf[i, pl.ds(256, 16)] = tl[i, pl.ds(_TAIL_IN, 16)]
        buf[i, pl.ds(272, 16)] = tl[i, pl.ds(_TAIL_IN + 16, 16)]
        rows = jnp.full((16,), i, jnp.int32)
        plsc.store_scatter(buf, [rows, cols], tl[i, pl.ds(_TAIL_IN + 28, 16)])
        return ()
      lax.fori_loop(0, _S, row, ())
      pltpu.sync_copy(buf, out_hbm.at[base + j])

    start(0, buf0, tl0, sem0)
    start(1, buf1, tl1, sem1)

    def body(i, _):
      j0 = 2 * i
      j1 = 2 * i + 1

      wait(buf0, tl0, sem0)
      compact_and_store(j0, buf0, tl0)

      @pl.when(j0 + 2 < _ROWS_W)
      def _():
        start(j0 + 2, buf0, tl0, sem0)

      wait(buf1, tl1, sem1)
      compact_and_store(j1, buf1, tl1)

      @pl.when(j1 + 2 < _ROWS_W)
      def _():
        start(j1 + 2, buf1, tl1, sem1)

      return ()

    lax.fori_loop(0, _ROWS_W // 2, body, ())

  return gather_kernel


_gather = _make_gather()


def kernel(x, table):
  idx = x.astype(jnp.int32)
  tail = table[:, _TAIL_OFF:]
  return _gather(idx, table, tail)


# speedup vs baseline: 1.3367x; 1.0019x over previous
import functools

import jax
import jax.numpy as jnp
from jax import lax
from jax.experimental import pallas as pl
from jax.experimental.pallas import tpu as pltpu
from jax.experimental.pallas import tpu_sc as plsc

_NUM_CORES = 2
_NUM_SUBCORES = 16
_NW = _NUM_CORES * _NUM_SUBCORES

_A = 4096
_S = 50
_D = 300
_ROWS_W = _A // _NW
_TAIL_OFF = 172
_TAIL_IN = 256 - _TAIL_OFF
_NB = 4


def _make_gather():
  mesh = plsc.VectorSubcoreMesh(core_axis_name="c", subcore_axis_name="s")

  @functools.partial(
      pl.kernel,
      mesh=mesh,
      compiler_params=pltpu.CompilerParams(needs_layout_passes=False),
      out_type=jax.ShapeDtypeStruct((_A, _S, _D), jnp.float32),
      scratch_types=[
          pltpu.VMEM((_ROWS_W, _S), jnp.int32),
          [pltpu.VMEM((_S, _D), jnp.float32) for _ in range(_NB)],
          [pltpu.VMEM((_S, 128), jnp.float32) for _ in range(_NB)],
          [pltpu.SemaphoreType.DMA for _ in range(_NB)],
          [pltpu.SemaphoreType.DMA for _ in range(_NB)],
      ],
  )
  def gather_kernel(idx_hbm, table_hbm, tail_hbm, out_hbm,
                    idx_v, bufs, tls, in_sems, out_sems):
    wid = lax.axis_index("s") * _NUM_CORES + lax.axis_index("c")
    base = wid * _ROWS_W

    pltpu.sync_copy(idx_hbm.at[pl.ds(base, _ROWS_W)], idx_v)

    def start_in(j, b):
      idx = idx_v.at[j]
      pltpu.make_async_copy(
          table_hbm.at[idx, pl.ds(0, 128)],
          bufs[b].at[:, pl.ds(0, 128)], in_sems[b]).start()
      pltpu.make_async_copy(
          table_hbm.at[idx, pl.ds(128, 128)],
          bufs[b].at[:, pl.ds(128, 128)], in_sems[b]).start()
      pltpu.make_async_copy(tail_hbm.at[idx], tls[b], in_sems[b]).start()

    def wait_in(b):
      pltpu.make_async_copy(
          table_hbm.at[idx_v.at[0], pl.ds(0, 128)],
          bufs[b].at[:, pl.ds(0, 128)], in_sems[b]).wait()
      pltpu.make_async_copy(
          table_hbm.at[idx_v.at[0], pl.ds(128, 128)],
          bufs[b].at[:, pl.ds(128, 128)], in_sems[b]).wait()
      pltpu.make_async_copy(tail_hbm.at[idx_v.at[0]], tls[b], in_sems[b]).wait()

    def start_out(j, b):
      pltpu.make_async_copy(bufs[b], out_hbm.at[base + j], out_sems[b]).start()

    def wait_out(b):
      pltpu.make_async_copy(bufs[b], out_hbm.at[base], out_sems[b]).wait()

    cols = 284 + lax.iota(jnp.int32, 16)

    def compact(b):
      buf, tl = bufs[b], tls[b]
      def row(i, _):
        buf[i, pl.ds(256, 16)] = tl[i, pl.ds(_TAIL_IN, 16)]
        buf[i, pl.ds(272, 16)] = tl[i, pl.ds(_TAIL_IN + 16, 16)]
        rows = jnp.full((16,), i, jnp.int32)
        plsc.store_scatter(buf, [rows, cols], tl[i, pl.ds(_TAIL_IN + 28, 16)])
        return ()
      lax.fori_loop(0, _S, row, ())

    start_in(0, 0)
    start_in(1, 1)

    def iteration(j, b):
      wait_in(b)
      compact(b)
      start_out(j, b)

      jn = j + 2
      bn = (b + 2) % _NB

      @pl.when(jn < _ROWS_W)
      def _():
        @pl.when(jn >= _NB)
        def _():
          wait_out(bn)
        start_in(jn, bn)

    def body(i, _):
      for b in range(_NB):
        iteration(_NB * i + b, b)
      return ()

    lax.fori_loop(0, _ROWS_W // _NB, body, ())

    for b in range(_NB):
      wait_out(b)

  return gather_kernel


_gather = _make_gather()


def kernel(x, table):
  idx = x.astype(jnp.int32)
  tail = table[:, _TAIL_OFF:]
  return _gather(idx, table, tail)
